# baseline (speedup 1.0000x reference)
import functools

import jax
import jax.numpy as jnp
from jax import lax
from jax.experimental import pallas as pl
from jax.experimental.pallas import tpu as pltpu
from jax.experimental.pallas import tpu_sc as plsc

N = 10000
E = 320000
D = 128
DH = D // 2

NC = 2
NS = 16
NW = NC * NS
CHUNK = 128
NCH = 158
EPAD = NS * NCH * CHUNK
NPAIR = NCH // 2

NP = 10240
SP = NP // NS
STRIPE = 624
REM = N - NS * STRIPE

DGRP = 8
DNCH = 80
DEPAD = NW * DNCH * CHUNK
RB = 1000


def _sc_mesh():
    return plsc.VectorSubcoreMesh(
        core_axis_name="c", subcore_axis_name="s", num_cores=NC, num_subcores=NS
    )


def _segsum_body(y_hbm, src_hbm, dst_hbm, zrows_hbm, out_hbm,
                 acc_sh, y_sh, isrc0, isrc1, idst0, idst1, rows0, rows1,
                 sem_i0, sem_i1, sem_d0, sem_d1, sem_g0, sem_g1,
                 sem_s0, sem_s1):
    c = lax.axis_index("c")
    s = lax.axis_index("s")

    pltpu.sync_copy(y_hbm.at[c, pl.ds(s * STRIPE, STRIPE)],
                    y_sh.at[pl.ds(s * STRIPE, STRIPE)])
    pltpu.sync_copy(zrows_hbm, acc_sh.at[pl.ds(s * SP, SP)])

    @pl.when(s == 0)
    def _stage_rem():
        pltpu.sync_copy(y_hbm.at[c, pl.ds(NS * STRIPE, REM)],
                        y_sh.at[pl.ds(NS * STRIPE, REM)])

    plsc.subcore_barrier()

    def start_iload(j, ibuf, sem):
        pltpu.async_copy(src_hbm.at[s, j], ibuf, sem)

    def wait_iload(j, ibuf, sem):
        pltpu.make_async_copy(src_hbm.at[s, j], ibuf, sem).wait()

    def start_dload(j, dbuf, sem):
        pltpu.async_copy(dst_hbm.at[s, j], dbuf, sem)

    def wait_dload(j, dbuf, sem):
        pltpu.make_async_copy(dst_hbm.at[s, j], dbuf, sem).wait()

    def start_gather(ibuf, buf, sem):
        pltpu.async_copy(y_sh.at[ibuf], buf, sem)

    def wait_gather(ibuf, buf, sem):
        pltpu.make_async_copy(y_sh.at[ibuf], buf, sem).wait()

    def start_scatter(dbuf, buf, sem):
        pltpu.async_copy(buf, acc_sh.at[dbuf], sem, add=True)

    def wait_scatter(dbuf, buf, sem):
        pltpu.make_async_copy(buf, acc_sh.at[dbuf], sem).wait()

    start_iload(0, isrc0, sem_i0)
    start_dload(0, idst0, sem_d0)
    wait_iload(0, isrc0, sem_i0)
    start_gather(isrc0, rows0, sem_g0)
    start_iload(1, isrc1, sem_i1)
    start_dload(1, idst1, sem_d1)

    def body(i, carry):
        a = 2 * i
        b = a + 1

        wait_iload(b, isrc1, sem_i1)
        start_gather(isrc1, rows1, sem_g1)
        wait_gather(isrc0, rows0, sem_g0)
        wait_dload(a, idst0, sem_d0)
        start_scatter(idst0, rows0, sem_s0)
        wait_scatter(idst0, rows0, sem_s0)

        @pl.when(i < NPAIR - 1)
        def _next0():
            start_iload(a + 2, isrc0, sem_i0)
            start_dload(a + 2, idst0, sem_d0)
            wait_iload(a + 2, isrc0, sem_i0)
            start_gather(isrc0, rows0, sem_g0)

        wait_gather(isrc1, rows1, sem_g1)
        wait_dload(b, idst1, sem_d1)
        start_scatter(idst1, rows1, sem_s1)
        wait_scatter(idst1, rows1, sem_s1)

        @pl.when(i < NPAIR - 1)
        def _next1():
            start_iload(b + 2, isrc1, sem_i1)
            start_dload(b + 2, idst1, sem_d1)

        return carry

    lax.fori_loop(0, NPAIR, body, 0)

    plsc.subcore_barrier()
    pltpu.sync_copy(acc_sh.at[pl.ds(s * STRIPE, STRIPE)],
                    out_hbm.at[c, pl.ds(s * STRIPE, STRIPE)])

    @pl.when(s == 0)
    def _write_rem():
        pltpu.sync_copy(acc_sh.at[pl.ds(NS * STRIPE, REM)],
                        out_hbm.at[c, pl.ds(NS * STRIPE, REM)])


@functools.cache
def _segsum_kernel():
    return pl.kernel(
        _segsum_body,
        out_type=jax.ShapeDtypeStruct((NC, N, DH), jnp.float32),
        mesh=_sc_mesh(),
        compiler_params=pltpu.CompilerParams(use_tc_tiling_on_sc=False),
        scratch_types=[
            pltpu.VMEM_SHARED((NP, DH), jnp.float32),
            pltpu.VMEM_SHARED((N, DH), jnp.float32),
            pltpu.VMEM((CHUNK,), jnp.int32),
            pltpu.VMEM((CHUNK,), jnp.int32),
            pltpu.VMEM((CHUNK,), jnp.int32),
            pltpu.VMEM((CHUNK,), jnp.int32),
            pltpu.VMEM((CHUNK, DH), jnp.float32),
            pltpu.VMEM((CHUNK, DH), jnp.float32),
            pltpu.SemaphoreType.DMA,
            pltpu.SemaphoreType.DMA,
            pltpu.SemaphoreType.DMA,
            pltpu.SemaphoreType.DMA,
            pltpu.SemaphoreType.DMA,
            pltpu.SemaphoreType.DMA,
            pltpu.SemaphoreType.DMA,
            pltpu.SemaphoreType.DMA,
        ],
    )


def _degree_body(dst_hbm, zeros_hbm, ones_hbm, out_hbm,
                 deg_sh, idst, ones_v, sem):
    c = lax.axis_index("c")
    s = lax.axis_index("s")
    wid = c * NS + s

    pltpu.sync_copy(dst_hbm.at[wid], idst)
    pltpu.sync_copy(zeros_hbm, deg_sh.at[pl.ds(s * SP, SP)])
    pltpu.sync_copy(ones_hbm, ones_v)
    plsc.subcore_barrier()

    def fire(j):
        pltpu.async_copy(ones_v, deg_sh.at[idst.at[j]], sem, add=True)

    def drain(j):
        pltpu.make_async_copy(ones_v, deg_sh.at[idst.at[j]], sem).wait()

    def body(g, carry):
        for k in range(DGRP):
            fire(g * DGRP + k)

        @pl.when(g > 0)
        def _drain_prev():
            for k in range(DGRP):
                drain((g - 1) * DGRP + k)

        return carry

    lax.fori_loop(0, DNCH // DGRP, body, 0)
    for k in range(DGRP):
        drain(DNCH - DGRP + k)

    plsc.subcore_barrier()
    pltpu.sync_copy(deg_sh.at[pl.ds(s * SP, SP)],
                    out_hbm.at[pl.ds(c * NP + s * SP, SP)])


@functools.cache
def _degree_kernel():
    return pl.kernel(
        _degree_body,
        out_type=jax.ShapeDtypeStruct((NC * NP,), jnp.float32),
        mesh=_sc_mesh(),
        scratch_types=[
            pltpu.VMEM_SHARED((NP,), jnp.float32),
            pltpu.VMEM((DNCH, CHUNK), jnp.int32),
            pltpu.VMEM((CHUNK,), jnp.float32),
            pltpu.SemaphoreType.DMA,
        ],
    )


def _tc_dis_body(hist_ref, dis_ref):
    deg = hist_ref[0] + hist_ref[1] + 1.0
    dis_ref[...] = lax.rsqrt(deg)


def _tc_dis(hist):
    return pl.pallas_call(
        _tc_dis_body,
        out_shape=jax.ShapeDtypeStruct((NP // 128, 128), jnp.float32),
    )(hist)


def _tc_first_body(dis_ref, x_ref, w_ref, y_ref):
    xw = jnp.dot(x_ref[...], w_ref[...], preferred_element_type=jnp.float32)
    y = xw * dis_ref[...]
    y_ref[0] = y[:, :DH]
    y_ref[1] = y[:, DH:]


def _tc_first(dis, x, W1):
    return pl.pallas_call(
        _tc_first_body,
        grid=(N // RB,),
        in_specs=[
            pl.BlockSpec((RB, 1), lambda i: (i, 0)),
            pl.BlockSpec((RB, D), lambda i: (i, 0)),
            pl.BlockSpec((D, D), lambda i: (0, 0)),
        ],
        out_specs=pl.BlockSpec((NC, RB, DH), lambda i: (0, i, 0)),
        out_shape=jax.ShapeDtypeStruct((NC, N, DH), jnp.float32),
    )(dis, x, W1)


def _tc_mid_body(dis_ref, acc_ref, y1_ref, b1_ref, w2_ref, y2_ref):
    dis = dis_ref[...]
    t = jnp.concatenate([acc_ref[0] + y1_ref[0], acc_ref[1] + y1_ref[1]],
                        axis=-1)
    h = dis * t + b1_ref[...]
    h = jnp.maximum(h, 0.0)
    hw = jnp.dot(h, w2_ref[...], preferred_element_type=jnp.float32)
    y2 = hw * dis
    y2_ref[0] = y2[:, :DH]
    y2_ref[1] = y2[:, DH:]


def _tc_mid(dis, acc, y1, b1, W2):
    return pl.pallas_call(
        _tc_mid_body,
        grid=(N // RB,),
        in_specs=[
            pl.BlockSpec((RB, 1), lambda i: (i, 0)),
            pl.BlockSpec((NC, RB, DH), lambda i: (0, i, 0)),
            pl.BlockSpec((NC, RB, DH), lambda i: (0, i, 0)),
            pl.BlockSpec((1, D), lambda i: (0, 0)),
            pl.BlockSpec((D, D), lambda i: (0, 0)),
        ],
        out_specs=pl.BlockSpec((NC, RB, DH), lambda i: (0, i, 0)),
        out_shape=jax.ShapeDtypeStruct((NC, N, DH), jnp.float32),
    )(dis, acc, y1, b1, W2)


def _tc_final_body(dis_ref, acc_ref, y2_ref, b2_ref, out_ref):
    t = jnp.concatenate([acc_ref[0] + y2_ref[0], acc_ref[1] + y2_ref[1]],
                        axis=-1)
    out_ref[...] = dis_ref[...] * t + b2_ref[...]


def _tc_final(dis, acc, y2, b2):
    return pl.pallas_call(
        _tc_final_body,
        grid=(N // RB,),
        in_specs=[
            pl.BlockSpec((RB, 1), lambda i: (i, 0)),
            pl.BlockSpec((NC, RB, DH), lambda i: (0, i, 0)),
            pl.BlockSpec((NC, RB, DH), lambda i: (0, i, 0)),
            pl.BlockSpec((1, D), lambda i: (0, 0)),
        ],
        out_specs=pl.BlockSpec((RB, D), lambda i: (i, 0)),
        out_shape=jax.ShapeDtypeStruct((N, D), jnp.float32),
    )(dis, acc, y2, b2)


def kernel(x, edge_index, W1, b1, W2, b2):
    src = edge_index[0]
    dst = edge_index[1]

    pad = EPAD - E
    pad_ar = jnp.arange(pad, dtype=jnp.int32)
    src3 = jnp.concatenate([src, pad_ar % N]).reshape(NS, NCH, CHUNK)
    dst3 = jnp.concatenate([dst, N + pad_ar % (NP - N)]).reshape(NS, NCH, CHUNK)

    dpad = DEPAD - E
    dpad_ar = jnp.arange(dpad, dtype=jnp.int32)
    dstd = jnp.concatenate([dst, N + dpad_ar % (NP - N)]).reshape(
        NW, DNCH, CHUNK)

    zrows = jnp.zeros((SP, DH), jnp.float32)
    zdeg = jnp.zeros((SP,), jnp.float32)
    ones = jnp.ones((CHUNK,), jnp.float32)

    hist = _degree_kernel()(dstd, zdeg, ones)
    dis_pad = _tc_dis(hist.reshape(NC, NP // 128, 128))
    dis = dis_pad.reshape(NP, 1)[:N]

    y1 = _tc_first(dis, x, W1)
    acc1 = _segsum_kernel()(y1, src3, dst3, zrows)
    y2 = _tc_mid(dis, acc1, y1, b1.reshape(1, D), W2)
    acc2 = _segsum_kernel()(y2, src3, dst3, zrows)
    out = _tc_final(dis, acc2, y2, b2.reshape(1, D))
    return out


# speedup vs baseline: 1.7287x; 1.7287x over previous
import functools

import jax
import jax.numpy as jnp
from jax import lax
from jax.experimental import pallas as pl
from jax.experimental.pallas import tpu as pltpu
from jax.experimental.pallas import tpu_sc as plsc

N = 10000
E = 320000
D = 128

NC = 2
NS = 16
NW = NC * NS
CHUNK = 128
NCH = 80
EPAD = NW * NCH * CHUNK
NPAIR = NCH // 2

NP = 10240
SP = NP // NS
STRIPE = 624
REM = N - NS * STRIPE

DGRP = 8
RB = 1000


def _sc_mesh():
    return plsc.VectorSubcoreMesh(
        core_axis_name="c", subcore_axis_name="s", num_cores=NC, num_subcores=NS
    )


def _segsum_body(y_hbm, src_hbm, dst_hbm, zrows_hbm, out_hbm,
                 acc_sh, idst, isrc0, isrc1, rows0, rows1,
                 sem_i0, sem_i1, sem_g0, sem_g1, sem_s0, sem_s1):
    c = lax.axis_index("c")
    s = lax.axis_index("s")
    wid = c * NS + s

    pltpu.sync_copy(dst_hbm.at[wid], idst)
    pltpu.sync_copy(zrows_hbm, acc_sh.at[pl.ds(s * SP, SP)])
    plsc.subcore_barrier()

    def start_iload(j, ibuf, sem):
        pltpu.async_copy(src_hbm.at[wid, j], ibuf, sem)

    def wait_iload(j, ibuf, sem):
        pltpu.make_async_copy(src_hbm.at[wid, j], ibuf, sem).wait()

    def start_gather(ibuf, buf, sem):
        pltpu.async_copy(y_hbm.at[ibuf], buf, sem)

    def wait_gather(ibuf, buf, sem):
        pltpu.make_async_copy(y_hbm.at[ibuf], buf, sem).wait()

    def start_scatter(j, buf, sem):
        pltpu.async_copy(buf, acc_sh.at[idst.at[j]], sem, add=True)

    def wait_scatter(j, buf, sem):
        pltpu.make_async_copy(buf, acc_sh.at[idst.at[j]], sem).wait()

    start_iload(0, isrc0, sem_i0)
    wait_iload(0, isrc0, sem_i0)
    start_gather(isrc0, rows0, sem_g0)
    start_iload(1, isrc1, sem_i1)

    def body(i, carry):
        a = 2 * i
        b = a + 1

        @pl.when(i > 0)
        def _drain_s_prev():
            wait_scatter(a - 1, rows1, sem_s1)

        wait_iload(b, isrc1, sem_i1)
        start_gather(isrc1, rows1, sem_g1)
        wait_gather(isrc0, rows0, sem_g0)

        @pl.when(i < NPAIR - 1)
        def _il_next0():
            start_iload(a + 2, isrc0, sem_i0)

        start_scatter(a, rows0, sem_s0)
        wait_scatter(a, rows0, sem_s0)

        @pl.when(i < NPAIR - 1)
        def _g_next0():
            wait_iload(a + 2, isrc0, sem_i0)
            start_gather(isrc0, rows0, sem_g0)

        wait_gather(isrc1, rows1, sem_g1)

        @pl.when(i < NPAIR - 1)
        def _il_next1():
            start_iload(b + 2, isrc1, sem_i1)

        start_scatter(b, rows1, sem_s1)
        return carry

    lax.fori_loop(0, NPAIR, body, 0)
    wait_scatter(NCH - 1, rows1, sem_s1)

    plsc.subcore_barrier()
    pltpu.sync_copy(acc_sh.at[pl.ds(s * STRIPE, STRIPE)],
                    out_hbm.at[c, pl.ds(s * STRIPE, STRIPE)])

    @pl.when(s == 0)
    def _write_rem():
        pltpu.sync_copy(acc_sh.at[pl.ds(NS * STRIPE, REM)],
                        out_hbm.at[c, pl.ds(NS * STRIPE, REM)])


@functools.cache
def _segsum_kernel():
    return pl.kernel(
        _segsum_body,
        out_type=jax.ShapeDtypeStruct((NC, N, D), jnp.float32),
        mesh=_sc_mesh(),
        scratch_types=[
            pltpu.VMEM_SHARED((NP, D), jnp.float32),
            pltpu.VMEM((NCH, CHUNK), jnp.int32),
            pltpu.VMEM((CHUNK,), jnp.int32),
            pltpu.VMEM((CHUNK,), jnp.int32),
            pltpu.VMEM((CHUNK, D), jnp.float32),
            pltpu.VMEM((CHUNK, D), jnp.float32),
            pltpu.SemaphoreType.DMA,
            pltpu.SemaphoreType.DMA,
            pltpu.SemaphoreType.DMA,
            pltpu.SemaphoreType.DMA,
            pltpu.SemaphoreType.DMA,
            pltpu.SemaphoreType.DMA,
        ],
    )


def _degree_body(dst_hbm, zeros_hbm, ones_hbm, out_hbm,
                 deg_sh, idst, ones_v, sem):
    c = lax.axis_index("c")
    s = lax.axis_index("s")
    wid = c * NS + s

    pltpu.sync_copy(dst_hbm.at[wid], idst)
    pltpu.sync_copy(zeros_hbm, deg_sh.at[pl.ds(s * SP, SP)])
    pltpu.sync_copy(ones_hbm, ones_v)
    plsc.subcore_barrier()

    def fire(j):
        pltpu.async_copy(ones_v, deg_sh.at[idst.at[j]], sem, add=True)

    def drain(j):
        pltpu.make_async_copy(ones_v, deg_sh.at[idst.at[j]], sem).wait()

    def body(g, carry):
        for k in range(DGRP):
            fire(g * DGRP + k)

        @pl.when(g > 0)
        def _drain_prev():
            for k in range(DGRP):
                drain((g - 1) * DGRP + k)

        return carry

    lax.fori_loop(0, NCH // DGRP, body, 0)
    for k in range(DGRP):
        drain(NCH - DGRP + k)

    plsc.subcore_barrier()
    pltpu.sync_copy(deg_sh.at[pl.ds(s * SP, SP)],
                    out_hbm.at[pl.ds(c * NP + s * SP, SP)])


@functools.cache
def _degree_kernel():
    return pl.kernel(
        _degree_body,
        out_type=jax.ShapeDtypeStruct((NC * NP,), jnp.float32),
        mesh=_sc_mesh(),
        scratch_types=[
            pltpu.VMEM_SHARED((NP,), jnp.float32),
            pltpu.VMEM((NCH, CHUNK), jnp.int32),
            pltpu.VMEM((CHUNK,), jnp.float32),
            pltpu.SemaphoreType.DMA,
        ],
    )


def _tc_dis_body(hist_ref, dis_ref):
    deg = hist_ref[0] + hist_ref[1] + 1.0
    dis_ref[...] = lax.rsqrt(deg)


def _tc_dis(hist):
    return pl.pallas_call(
        _tc_dis_body,
        out_shape=jax.ShapeDtypeStruct((NP // 128, 128), jnp.float32),
    )(hist)


def _tc_first_body(dis_ref, x_ref, w_ref, y_ref):
    xw = jnp.dot(x_ref[...], w_ref[...], preferred_element_type=jnp.float32)
    y_ref[...] = xw * dis_ref[...]


def _tc_first(dis, x, W1):
    return pl.pallas_call(
        _tc_first_body,
        grid=(N // RB,),
        in_specs=[
            pl.BlockSpec((RB, 1), lambda i: (i, 0)),
            pl.BlockSpec((RB, D), lambda i: (i, 0)),
            pl.BlockSpec((D, D), lambda i: (0, 0)),
        ],
        out_specs=pl.BlockSpec((RB, D), lambda i: (i, 0)),
        out_shape=jax.ShapeDtypeStruct((N, D), jnp.float32),
    )(dis, x, W1)


def _tc_mid_body(dis_ref, acc_ref, y1_ref, b1_ref, w2_ref, y2_ref):
    dis = dis_ref[...]
    h = dis * (acc_ref[0] + acc_ref[1] + y1_ref[...]) + b1_ref[...]
    h = jnp.maximum(h, 0.0)
    hw = jnp.dot(h, w2_ref[...], preferred_element_type=jnp.float32)
    y2_ref[...] = hw * dis


def _tc_mid(dis, acc, y1, b1, W2):
    return pl.pallas_call(
        _tc_mid_body,
        grid=(N // RB,),
        in_specs=[
            pl.BlockSpec((RB, 1), lambda i: (i, 0)),
            pl.BlockSpec((NC, RB, D), lambda i: (0, i, 0)),
            pl.BlockSpec((RB, D), lambda i: (i, 0)),
            pl.BlockSpec((1, D), lambda i: (0, 0)),
            pl.BlockSpec((D, D), lambda i: (0, 0)),
        ],
        out_specs=pl.BlockSpec((RB, D), lambda i: (i, 0)),
        out_shape=jax.ShapeDtypeStruct((N, D), jnp.float32),
    )(dis, acc, y1, b1, W2)


def _tc_final_body(dis_ref, acc_ref, y2_ref, b2_ref, out_ref):
    out_ref[...] = (
        dis_ref[...] * (acc_ref[0] + acc_ref[1] + y2_ref[...]) + b2_ref[...]
    )


def _tc_final(dis, acc, y2, b2):
    return pl.pallas_call(
        _tc_final_body,
        grid=(N // RB,),
        in_specs=[
            pl.BlockSpec((RB, 1), lambda i: (i, 0)),
            pl.BlockSpec((NC, RB, D), lambda i: (0, i, 0)),
            pl.BlockSpec((RB, D), lambda i: (i, 0)),
            pl.BlockSpec((1, D), lambda i: (0, 0)),
        ],
        out_specs=pl.BlockSpec((RB, D), lambda i: (i, 0)),
        out_shape=jax.ShapeDtypeStruct((N, D), jnp.float32),
    )(dis, acc, y2, b2)


def kernel(x, edge_index, W1, b1, W2, b2):
    src = edge_index[0]
    dst = edge_index[1]

    pad = EPAD - E
    pad_ar = jnp.arange(pad, dtype=jnp.int32)
    src3 = jnp.concatenate([src, pad_ar % N]).reshape(NW, NCH, CHUNK)
    dst3 = jnp.concatenate([dst, N + pad_ar % (NP - N)]).reshape(NW, NCH, CHUNK)

    zrows = jnp.zeros((SP, D), jnp.float32)
    zdeg = jnp.zeros((SP,), jnp.float32)
    ones = jnp.ones((CHUNK,), jnp.float32)

    hist = _degree_kernel()(dst3, zdeg, ones)
    dis_pad = _tc_dis(hist.reshape(NC, NP // 128, 128))
    dis = dis_pad.reshape(NP, 1)[:N]

    y1 = _tc_first(dis, x, W1)
    acc1 = _segsum_kernel()(y1, src3, dst3, zrows)
    y2 = _tc_mid(dis, acc1, y1, b1.reshape(1, D), W2)
    acc2 = _segsum_kernel()(y2, src3, dst3, zrows)
    out = _tc_final(dis, acc2, y2, b2.reshape(1, D))
    return out


# speedup vs baseline: 1.7301x; 1.0008x over previous
import functools

import jax
import jax.numpy as jnp
from jax import lax
from jax.experimental import pallas as pl
from jax.experimental.pallas import tpu as pltpu
from jax.experimental.pallas import tpu_sc as plsc

N = 10000
E = 320000
D = 128

NC = 2
NS = 16
NW = NC * NS
CHUNK = 128
NCH = 80
EPAD = NW * NCH * CHUNK
NPAIR = NCH // 2

NP = 10240
SP = NP // NS
STRIPE = 624
REM = N - NS * STRIPE

DGRP = 8
RB = 1000


def _sc_mesh():
    return plsc.VectorSubcoreMesh(
        core_axis_name="c", subcore_axis_name="s", num_cores=NC, num_subcores=NS
    )


def _segsum_body(y_hbm, src_hbm, dst_hbm, zrows_hbm, out_hbm,
                 acc_sh, idst, isrc0, isrc1, rows0, rows1,
                 sem_i0, sem_i1, sem_g0, sem_g1, sem_s0, sem_s1):
    c = lax.axis_index("c")
    s = lax.axis_index("s")
    wid = c * NS + s

    pltpu.sync_copy(dst_hbm.at[wid], idst)
    pltpu.sync_copy(zrows_hbm, acc_sh.at[pl.ds(s * SP, SP)])
    plsc.subcore_barrier()

    def start_iload(j, ibuf, sem):
        pltpu.async_copy(src_hbm.at[wid, j], ibuf, sem)

    def wait_iload(j, ibuf, sem):
        pltpu.make_async_copy(src_hbm.at[wid, j], ibuf, sem).wait()

    H = CHUNK // 2

    def start_gather(ibuf, buf, sem):
        pltpu.async_copy(y_hbm.at[ibuf.at[pl.ds(0, H)]], buf.at[pl.ds(0, H)], sem)
        pltpu.async_copy(y_hbm.at[ibuf.at[pl.ds(H, H)]], buf.at[pl.ds(H, H)], sem)

    def wait_gather(ibuf, buf, sem):
        pltpu.make_async_copy(y_hbm.at[ibuf.at[pl.ds(0, H)]], buf.at[pl.ds(0, H)], sem).wait()
        pltpu.make_async_copy(y_hbm.at[ibuf.at[pl.ds(H, H)]], buf.at[pl.ds(H, H)], sem).wait()

    def start_scatter(j, buf, sem):
        pltpu.async_copy(buf, acc_sh.at[idst.at[j]], sem, add=True)

    def wait_scatter(j, buf, sem):
        pltpu.make_async_copy(buf, acc_sh.at[idst.at[j]], sem).wait()

    start_iload(0, isrc0, sem_i0)
    wait_iload(0, isrc0, sem_i0)
    start_gather(isrc0, rows0, sem_g0)
    start_iload(1, isrc1, sem_i1)

    def body(i, carry):
        a = 2 * i
        b = a + 1

        @pl.when(i > 0)
        def _drain_s_prev():
            wait_scatter(a - 1, rows1, sem_s1)

        wait_iload(b, isrc1, sem_i1)
        start_gather(isrc1, rows1, sem_g1)
        wait_gather(isrc0, rows0, sem_g0)

        @pl.when(i < NPAIR - 1)
        def _il_next0():
            start_iload(a + 2, isrc0, sem_i0)

        start_scatter(a, rows0, sem_s0)
        wait_scatter(a, rows0, sem_s0)

        @pl.when(i < NPAIR - 1)
        def _g_next0():
            wait_iload(a + 2, isrc0, sem_i0)
            start_gather(isrc0, rows0, sem_g0)

        wait_gather(isrc1, rows1, sem_g1)

        @pl.when(i < NPAIR - 1)
        def _il_next1():
            start_iload(b + 2, isrc1, sem_i1)

        start_scatter(b, rows1, sem_s1)
        return carry

    lax.fori_loop(0, NPAIR, body, 0)
    wait_scatter(NCH - 1, rows1, sem_s1)

    plsc.subcore_barrier()
    pltpu.sync_copy(acc_sh.at[pl.ds(s * STRIPE, STRIPE)],
                    out_hbm.at[c, pl.ds(s * STRIPE, STRIPE)])

    @pl.when(s == 0)
    def _write_rem():
        pltpu.sync_copy(acc_sh.at[pl.ds(NS * STRIPE, REM)],
                        out_hbm.at[c, pl.ds(NS * STRIPE, REM)])


@functools.cache
def _segsum_kernel():
    return pl.kernel(
        _segsum_body,
        out_type=jax.ShapeDtypeStruct((NC, N, D), jnp.float32),
        mesh=_sc_mesh(),
        scratch_types=[
            pltpu.VMEM_SHARED((NP, D), jnp.float32),
            pltpu.VMEM((NCH, CHUNK), jnp.int32),
            pltpu.VMEM((CHUNK,), jnp.int32),
            pltpu.VMEM((CHUNK,), jnp.int32),
            pltpu.VMEM((CHUNK, D), jnp.float32),
            pltpu.VMEM((CHUNK, D), jnp.float32),
            pltpu.SemaphoreType.DMA,
            pltpu.SemaphoreType.DMA,
            pltpu.SemaphoreType.DMA,
            pltpu.SemaphoreType.DMA,
            pltpu.SemaphoreType.DMA,
            pltpu.SemaphoreType.DMA,
        ],
    )


def _degree_body(dst_hbm, zeros_hbm, ones_hbm, out_hbm,
                 deg_sh, idst, ones_v, sem):
    c = lax.axis_index("c")
    s = lax.axis_index("s")
    wid = c * NS + s

    pltpu.sync_copy(dst_hbm.at[wid], idst)
    pltpu.sync_copy(zeros_hbm, deg_sh.at[pl.ds(s * SP, SP)])
    pltpu.sync_copy(ones_hbm, ones_v)
    plsc.subcore_barrier()

    def fire(j):
        pltpu.async_copy(ones_v, deg_sh.at[idst.at[j]], sem, add=True)

    def drain(j):
        pltpu.make_async_copy(ones_v, deg_sh.at[idst.at[j]], sem).wait()

    def body(g, carry):
        for k in range(DGRP):
            fire(g * DGRP + k)

        @pl.when(g > 0)
        def _drain_prev():
            for k in range(DGRP):
                drain((g - 1) * DGRP + k)

        return carry

    lax.fori_loop(0, NCH // DGRP, body, 0)
    for k in range(DGRP):
        drain(NCH - DGRP + k)

    plsc.subcore_barrier()
    pltpu.sync_copy(deg_sh.at[pl.ds(s * SP, SP)],
                    out_hbm.at[pl.ds(c * NP + s * SP, SP)])


@functools.cache
def _degree_kernel():
    return pl.kernel(
        _degree_body,
        out_type=jax.ShapeDtypeStruct((NC * NP,), jnp.float32),
        mesh=_sc_mesh(),
        scratch_types=[
            pltpu.VMEM_SHARED((NP,), jnp.float32),
            pltpu.VMEM((NCH, CHUNK), jnp.int32),
            pltpu.VMEM((CHUNK,), jnp.float32),
            pltpu.SemaphoreType.DMA,
        ],
    )


def _tc_dis_body(hist_ref, dis_ref):
    deg = hist_ref[0] + hist_ref[1] + 1.0
    dis_ref[...] = lax.rsqrt(deg)


def _tc_dis(hist):
    return pl.pallas_call(
        _tc_dis_body,
        out_shape=jax.ShapeDtypeStruct((NP // 128, 128), jnp.float32),
    )(hist)


def _tc_first_body(dis_ref, x_ref, w_ref, y_ref):
    xw = jnp.dot(x_ref[...], w_ref[...], preferred_element_type=jnp.float32)
    y_ref[...] = xw * dis_ref[...]


def _tc_first(dis, x, W1):
    return pl.pallas_call(
        _tc_first_body,
        grid=(N // RB,),
        in_specs=[
            pl.BlockSpec((RB, 1), lambda i: (i, 0)),
            pl.BlockSpec((RB, D), lambda i: (i, 0)),
            pl.BlockSpec((D, D), lambda i: (0, 0)),
        ],
        out_specs=pl.BlockSpec((RB, D), lambda i: (i, 0)),
        out_shape=jax.ShapeDtypeStruct((N, D), jnp.float32),
    )(dis, x, W1)


def _tc_mid_body(dis_ref, acc_ref, y1_ref, b1_ref, w2_ref, y2_ref):
    dis = dis_ref[...]
    h = dis * (acc_ref[0] + acc_ref[1] + y1_ref[...]) + b1_ref[...]
    h = jnp.maximum(h, 0.0)
    hw = jnp.dot(h, w2_ref[...], preferred_element_type=jnp.float32)
    y2_ref[...] = hw * dis


def _tc_mid(dis, acc, y1, b1, W2):
    return pl.pallas_call(
        _tc_mid_body,
        grid=(N // RB,),
        in_specs=[
            pl.BlockSpec((RB, 1), lambda i: (i, 0)),
            pl.BlockSpec((NC, RB, D), lambda i: (0, i, 0)),
            pl.BlockSpec((RB, D), lambda i: (i, 0)),
            pl.BlockSpec((1, D), lambda i: (0, 0)),
            pl.BlockSpec((D, D), lambda i: (0, 0)),
        ],
        out_specs=pl.BlockSpec((RB, D), lambda i: (i, 0)),
        out_shape=jax.ShapeDtypeStruct((N, D), jnp.float32),
    )(dis, acc, y1, b1, W2)


def _tc_final_body(dis_ref, acc_ref, y2_ref, b2_ref, out_ref):
    out_ref[...] = (
        dis_ref[...] * (acc_ref[0] + acc_ref[1] + y2_ref[...]) + b2_ref[...]
    )


def _tc_final(dis, acc, y2, b2):
    return pl.pallas_call(
        _tc_final_body,
        grid=(N // RB,),
        in_specs=[
            pl.BlockSpec((RB, 1), lambda i: (i, 0)),
            pl.BlockSpec((NC, RB, D), lambda i: (0, i, 0)),
            pl.BlockSpec((RB, D), lambda i: (i, 0)),
            pl.BlockSpec((1, D), lambda i: (0, 0)),
        ],
        out_specs=pl.BlockSpec((RB, D), lambda i: (i, 0)),
        out_shape=jax.ShapeDtypeStruct((N, D), jnp.float32),
    )(dis, acc, y2, b2)


def kernel(x, edge_index, W1, b1, W2, b2):
    src = edge_index[0]
    dst = edge_index[1]

    pad = EPAD - E
    pad_ar = jnp.arange(pad, dtype=jnp.int32)
    src3 = jnp.concatenate([src, pad_ar % N]).reshape(NW, NCH, CHUNK)
    dst3 = jnp.concatenate([dst, N + pad_ar % (NP - N)]).reshape(NW, NCH, CHUNK)

    zrows = jnp.zeros((SP, D), jnp.float32)
    zdeg = jnp.zeros((SP,), jnp.float32)
    ones = jnp.ones((CHUNK,), jnp.float32)

    hist = _degree_kernel()(dst3, zdeg, ones)
    dis_pad = _tc_dis(hist.reshape(NC, NP // 128, 128))
    dis = dis_pad.reshape(NP, 1)[:N]

    y1 = _tc_first(dis, x, W1)
    acc1 = _segsum_kernel()(y1, src3, dst3, zrows)
    y2 = _tc_mid(dis, acc1, y1, b1.reshape(1, D), W2)
    acc2 = _segsum_kernel()(y2, src3, dst3, zrows)
    out = _tc_final(dis, acc2, y2, b2.reshape(1, D))
    return out
